# baseline (speedup 1.0000x reference)
import jax
import jax.numpy as jnp
from jax.experimental import pallas as pl
from jax.experimental.pallas import tpu as pltpu
from jax.experimental.pallas import tpu_sc as plsc

_BLOCK_ROWS = 2048


def _sc_gather_body(idx_hbm, emb_hbm, row_hbm, idx_smem, sem):
    @pl.when(jax.lax.axis_index("c") == 0)
    def _():
        pltpu.async_copy(idx_hbm, idx_smem, sem).wait()
        pltpu.async_copy(emb_hbm.at[pl.ds(idx_smem[0], 1)], row_hbm, sem).wait()


def _tc_add_body(row_ref, x_ref, o_ref):
    o_ref[...] = x_ref[...] + row_ref[...]


def kernel(x, rate_id, rate_embedding):
    b, s, d = x.shape
    rows = b * s
    x2 = x.reshape(rows, d)
    idx = jnp.asarray([rate_id], dtype=jnp.int32)

    row = pl.kernel(
        _sc_gather_body,
        out_type=jax.ShapeDtypeStruct((1, d), rate_embedding.dtype),
        mesh=plsc.ScalarSubcoreMesh(axis_name="c", num_cores=2),
        scratch_types=[
            pltpu.SMEM((1,), jnp.int32),
            pltpu.SemaphoreType.DMA,
        ],
    )(idx, rate_embedding)

    block = min(_BLOCK_ROWS, rows)
    out = pl.pallas_call(
        _tc_add_body,
        grid=(rows // block,),
        in_specs=[
            pl.BlockSpec((1, d), lambda i: (0, 0)),
            pl.BlockSpec((block, d), lambda i: (i, 0)),
        ],
        out_specs=pl.BlockSpec((block, d), lambda i: (i, 0)),
        out_shape=jax.ShapeDtypeStruct((rows, d), x.dtype),
        compiler_params=pltpu.CompilerParams(
            dimension_semantics=("arbitrary",),
        ),
    )(row, x2)
    return out.reshape(b, s, d)
